# Initial kernel scaffold; baseline (speedup 1.0000x reference)
import jax
import jax.numpy as jnp
from jax.experimental import pallas as pl


def kernel(hidden_states, text_end_position, multiway_split_position, text_table, image_table):
    raise NotImplementedError("write your pallas kernel here")


# speedup vs baseline: 1.5828x; 1.5828x over previous
import functools

import jax
import jax.numpy as jnp
from jax import lax
from jax.experimental import pallas as pl
from jax.experimental.pallas import tpu as pltpu
from jax.experimental.pallas import tpu_sc as plsc

D = 1024
S = 4096
R = 2 * S
NW = 32
HALF_PER_W = S // NW
CH = 64
NCH = HALF_PER_W // CH


@functools.partial(
    pl.kernel,
    mesh=plsc.VectorSubcoreMesh(core_axis_name="c", subcore_axis_name="s"),
    out_type=jax.ShapeDtypeStruct((R, D), jnp.float32),
    scratch_types=[
        pltpu.VMEM((CH,), jnp.int32),
        pltpu.VMEM((CH, D), jnp.float32),
        pltpu.SemaphoreType.DMA,
    ],
)
def _gather_kernel(idx_hbm, text_hbm, image_hbm, out_hbm, idx_v, rows_v, sem):
    wid = lax.axis_index("s") * 2 + lax.axis_index("c")
    base = wid * HALF_PER_W

    for half, table_hbm in ((0, text_hbm), (1, image_hbm)):
        for k in range(NCH):
            off = base + k * CH
            pltpu.sync_copy(idx_hbm.at[pl.ds(off, CH)], idx_v)
            pltpu.async_copy(table_hbm.at[idx_v], rows_v, sem).wait()
            pltpu.sync_copy(rows_v, out_hbm.at[pl.ds(half * S + off, CH)])


def kernel(hidden_states, text_end_position, multiway_split_position, text_table, image_table):
    del hidden_states, multiway_split_position
    idx = text_end_position.reshape(S).astype(jnp.int32)
    out = _gather_kernel(idx, text_table, image_table)
    return out.reshape(1, R, D)


# speedup vs baseline: 1.6686x; 1.0543x over previous
import functools

import jax
import jax.numpy as jnp
from jax import lax
from jax.experimental import pallas as pl
from jax.experimental.pallas import tpu as pltpu
from jax.experimental.pallas import tpu_sc as plsc

D = 1024
S = 4096
R = 2 * S
NW = 32
HALF_PER_W = S // NW
CH = 32
NCH = HALF_PER_W // CH
NT = 2 * NCH


@functools.partial(
    pl.kernel,
    mesh=plsc.VectorSubcoreMesh(core_axis_name="c", subcore_axis_name="s"),
    out_type=jax.ShapeDtypeStruct((R, D), jnp.float32),
    scratch_types=[
        pltpu.VMEM((HALF_PER_W,), jnp.int32),
        pltpu.VMEM((CH, D), jnp.float32),
        pltpu.VMEM((CH, D), jnp.float32),
        pltpu.SemaphoreType.DMA,
        pltpu.SemaphoreType.DMA,
        pltpu.SemaphoreType.DMA,
        pltpu.SemaphoreType.DMA,
    ],
)
def _gather_kernel(idx_hbm, text_hbm, image_hbm, out_hbm, idx_v,
                   buf_a, buf_b, sg_a, sg_b, ss_a, ss_b):
    wid = lax.axis_index("s") * 2 + lax.axis_index("c")
    base = wid * HALF_PER_W
    pltpu.sync_copy(idx_hbm.at[pl.ds(base, HALF_PER_W)], idx_v)

    bufs = (buf_a, buf_b)
    sg = (sg_a, sg_b)
    ss = (ss_a, ss_b)
    tables = (text_hbm, image_hbm)

    def start_gather(i):
        b = i % 2
        half, k = divmod(i, NCH)
        idx_slice = idx_v.at[pl.ds(k * CH, CH)]
        return pltpu.async_copy(tables[half].at[idx_slice], bufs[b], sg[b])

    def start_store(i):
        b = i % 2
        half, k = divmod(i, NCH)
        dst = out_hbm.at[pl.ds(half * S + base + k * CH, CH)]
        return pltpu.async_copy(bufs[b], dst, ss[b])

    g = [None] * NT
    s = [None] * NT
    g[0] = start_gather(0)
    for i in range(NT):
        if i + 1 < NT:
            if i >= 1:
                s[i - 1].wait()
            g[i + 1] = start_gather(i + 1)
        g[i].wait()
        s[i] = start_store(i)
    s[NT - 2].wait()
    s[NT - 1].wait()


def kernel(hidden_states, text_end_position, multiway_split_position, text_table, image_table):
    del hidden_states, multiway_split_position
    idx = text_end_position.reshape(S).astype(jnp.int32)
    out = _gather_kernel(idx, text_table, image_table)
    return out.reshape(1, R, D)
